# baseline (speedup 1.0000x reference)
import functools

import jax
import jax.numpy as jnp
from jax import lax
from jax.experimental import pallas as pl
from jax.experimental.pallas import tpu as pltpu
from jax.experimental.pallas import tpu_sc as plsc

B = 16384
F = 26
E = 255
P = 10
PE = 31
NTILES = 32
RPT = B // NTILES
NG = RPT // 16
UNROLL = 4

ESTRIDE = 272
PSTRIDE = 32

_mesh = plsc.VectorSubcoreMesh(core_axis_name="c", subcore_axis_name="s")


def _skew(a, n, stride):
    j = jnp.arange(n)
    out = jnp.zeros(a.shape[:-1] + (stride,), a.dtype)
    return out.at[..., j + (j // 16)].set(a)


@functools.partial(
    pl.kernel,
    mesh=_mesh,
    out_type=jax.ShapeDtypeStruct((B,), jnp.float32),
    compiler_params=pltpu.CompilerParams(needs_layout_passes=False),
    scratch_types=[
        pltpu.VMEM((F, RPT), jnp.float32),
        pltpu.VMEM((F * ESTRIDE,), jnp.float32),
        pltpu.VMEM((F * 256,), jnp.float32),
        pltpu.VMEM((P * 2 * PSTRIDE,), jnp.float32),
        pltpu.VMEM((P * 1024,), jnp.float32),
        pltpu.VMEM((2 * P, 16), jnp.int32),
        pltpu.VMEM((16,), jnp.float32),
        pltpu.VMEM((RPT,), jnp.float32),
        pltpu.SemaphoreType.DMA,
    ],
)
def _ebm_sc(xt_hbm, edges_hbm, v_hbm, pe_hbm, t2_hbm, pidx_hbm, bias_hbm,
            out_hbm,
            xt_v, edges_v, v_v, pe_v, t2_v, pidx_v, bias_v, out_v, sem):
    wid = lax.axis_index("s") * 2 + lax.axis_index("c")
    base = wid * RPT

    copies = [
        pltpu.async_copy(xt_hbm.at[:, pl.ds(base, RPT)], xt_v, sem),
        pltpu.async_copy(edges_hbm, edges_v, sem),
        pltpu.async_copy(v_hbm, v_v, sem),
        pltpu.async_copy(pe_hbm, pe_v, sem),
        pltpu.async_copy(t2_hbm, t2_v, sem),
        pltpu.async_copy(pidx_hbm, pidx_v, sem),
        pltpu.async_copy(bias_hbm, bias_v, sem),
    ]
    for c in copies:
        c.wait()

    lanes = lax.iota(jnp.int32, 16)
    zeros = jnp.zeros((16,), jnp.int32)
    bias_splat = bias_v[...]
    pid_l = [pidx_v[2 * p, :] for p in range(P)]
    pid_r = [pidx_v[2 * p + 1, :] for p in range(P)]

    def one_group(g):
        row0 = g * 16
        rowv = row0 + lanes
        acc = bias_splat
        for f in range(F):
            x = xt_v[f, pl.ds(row0, 16)]
            b = zeros
            for k in (128, 64, 32, 16, 8, 4, 2, 1):
                pos = b + (k - 1)
                e = plsc.load_gather(
                    edges_v, [pos + (pos >> 4) + f * ESTRIDE])
                b = b + jnp.where(e <= x, k, 0)
            acc = acc + plsc.load_gather(v_v, [b + f * 256])
        for p in range(P):
            xl = plsc.load_gather(xt_v, [pid_l[p], rowv])
            xr = plsc.load_gather(xt_v, [pid_r[p], rowv])
            bl = zeros
            br = zeros
            for k in (16, 8, 4, 2, 1):
                posl = bl + (k - 1)
                el = plsc.load_gather(
                    pe_v, [posl + (posl >> 4) + (2 * p) * PSTRIDE])
                bl = bl + jnp.where(el <= xl, k, 0)
                posr = br + (k - 1)
                er = plsc.load_gather(
                    pe_v, [posr + (posr >> 4) + (2 * p + 1) * PSTRIDE])
                br = br + jnp.where(er <= xr, k, 0)
            flat = (bl << 5) + br + p * 1024
            acc = acc + plsc.load_gather(t2_v, [flat])
        out_v[pl.ds(row0, 16)] = acc

    def body(i, carry):
        for u in range(UNROLL):
            one_group(i * UNROLL + u)
        return carry

    lax.fori_loop(0, NG // UNROLL, body, 0)
    pltpu.sync_copy(out_v, out_hbm.at[pl.ds(base, RPT)])


def kernel(X, edges, W, pair_edges, pair_tables, pair_idx, bias):
    xt = X.T
    edges_s = _skew(edges, E, ESTRIDE).reshape(-1)
    v_f = jnp.roll(W, -1, axis=1).reshape(-1)
    pe_s = _skew(pair_edges, PE, PSTRIDE).reshape(-1)
    t2_f = jnp.roll(jnp.roll(pair_tables, -1, axis=1), -1, axis=2).reshape(-1)
    pidx_s = jnp.broadcast_to(
        pair_idx.reshape(2 * P, 1).astype(jnp.int32), (2 * P, 16))
    bias16 = jnp.broadcast_to(bias.astype(jnp.float32), (16,))
    return _ebm_sc(xt, edges_s, v_f, pe_s, t2_f, pidx_s, bias16)


# speedup vs baseline: 1.4348x; 1.4348x over previous
import functools

import jax
import jax.numpy as jnp
from jax import lax
from jax.experimental import pallas as pl
from jax.experimental.pallas import tpu as pltpu
from jax.experimental.pallas import tpu_sc as plsc

B = 16384
F = 26
E = 255
P = 10
PE = 31
NTILES = 32
RPT = B // NTILES
NG = RPT // 16
UNROLL = 1

ESTRIDE = 272
PSTRIDE = 32

_mesh = plsc.VectorSubcoreMesh(core_axis_name="c", subcore_axis_name="s")


def _skew(a, n, stride):
    j = jnp.arange(n)
    out = jnp.zeros(a.shape[:-1] + (stride,), a.dtype)
    return out.at[..., j + (j // 16)].set(a)


@functools.partial(
    pl.kernel,
    mesh=_mesh,
    out_type=jax.ShapeDtypeStruct((B,), jnp.float32),
    compiler_params=pltpu.CompilerParams(needs_layout_passes=False),
    scratch_types=[
        pltpu.VMEM((F, RPT), jnp.float32),
        pltpu.VMEM((F * ESTRIDE,), jnp.float32),
        pltpu.VMEM((F * 256,), jnp.float32),
        pltpu.VMEM((P * 2 * PSTRIDE,), jnp.float32),
        pltpu.VMEM((P * 1024,), jnp.float32),
        pltpu.VMEM((2 * P, 16), jnp.int32),
        pltpu.VMEM((16,), jnp.float32),
        pltpu.VMEM((RPT,), jnp.float32),
        pltpu.SemaphoreType.DMA,
    ],
)
def _ebm_sc(xt_hbm, edges_hbm, v_hbm, pe_hbm, t2_hbm, pidx_hbm, bias_hbm,
            out_hbm,
            xt_v, edges_v, v_v, pe_v, t2_v, pidx_v, bias_v, out_v, sem):
    wid = lax.axis_index("s") * 2 + lax.axis_index("c")
    base = wid * RPT

    copies = [
        pltpu.async_copy(xt_hbm.at[:, pl.ds(base, RPT)], xt_v, sem),
        pltpu.async_copy(edges_hbm, edges_v, sem),
        pltpu.async_copy(v_hbm, v_v, sem),
        pltpu.async_copy(pe_hbm, pe_v, sem),
        pltpu.async_copy(t2_hbm, t2_v, sem),
        pltpu.async_copy(pidx_hbm, pidx_v, sem),
        pltpu.async_copy(bias_hbm, bias_v, sem),
    ]
    for c in copies:
        c.wait()

    lanes = lax.iota(jnp.int32, 16)
    zeros = jnp.zeros((16,), jnp.int32)
    bias_splat = bias_v[...]
    pid_l = [pidx_v[2 * p, :] for p in range(P)]
    pid_r = [pidx_v[2 * p + 1, :] for p in range(P)]

    def one_group(g):
        row0 = g * 16
        rowv = row0 + lanes
        acc = bias_splat
        for f in range(F):
            x = xt_v[f, pl.ds(row0, 16)]
            b = zeros
            for k in (128, 64, 32, 16, 8, 4, 2, 1):
                pos = b + (k - 1)
                e = plsc.load_gather(
                    edges_v, [pos + (pos >> 4) + f * ESTRIDE])
                b = b + jnp.where(e <= x, k, 0)
            acc = acc + plsc.load_gather(v_v, [b + f * 256])
        for p in range(P):
            xl = plsc.load_gather(xt_v, [pid_l[p], rowv])
            xr = plsc.load_gather(xt_v, [pid_r[p], rowv])
            bl = zeros
            br = zeros
            for k in (16, 8, 4, 2, 1):
                posl = bl + (k - 1)
                el = plsc.load_gather(
                    pe_v, [posl + (posl >> 4) + (2 * p) * PSTRIDE])
                bl = bl + jnp.where(el <= xl, k, 0)
                posr = br + (k - 1)
                er = plsc.load_gather(
                    pe_v, [posr + (posr >> 4) + (2 * p + 1) * PSTRIDE])
                br = br + jnp.where(er <= xr, k, 0)
            flat = (bl << 5) + br + p * 1024
            acc = acc + plsc.load_gather(t2_v, [flat])
        out_v[pl.ds(row0, 16)] = acc

    def body(i, carry):
        for u in range(UNROLL):
            one_group(i * UNROLL + u)
        return carry

    lax.fori_loop(0, NG // UNROLL, body, 0)
    pltpu.sync_copy(out_v, out_hbm.at[pl.ds(base, RPT)])


def kernel(X, edges, W, pair_edges, pair_tables, pair_idx, bias):
    xt = X.T
    edges_s = _skew(edges, E, ESTRIDE).reshape(-1)
    v_f = jnp.roll(W, -1, axis=1).reshape(-1)
    pe_s = _skew(pair_edges, PE, PSTRIDE).reshape(-1)
    t2_f = jnp.roll(jnp.roll(pair_tables, -1, axis=1), -1, axis=2).reshape(-1)
    pidx_s = jnp.broadcast_to(
        pair_idx.reshape(2 * P, 1).astype(jnp.int32), (2 * P, 16))
    bias16 = jnp.broadcast_to(bias.astype(jnp.float32), (16,))
    return _ebm_sc(xt, edges_s, v_f, pe_s, t2_f, pidx_s, bias16)


# speedup vs baseline: 1.4907x; 1.0389x over previous
import functools

import jax
import jax.numpy as jnp
from jax import lax
from jax.experimental import pallas as pl
from jax.experimental.pallas import tpu as pltpu
from jax.experimental.pallas import tpu_sc as plsc

B = 16384
F = 26
E = 255
P = 10
PE = 31
NTILES = 32
RPT = B // NTILES
NG = RPT // 16
UNROLL = 1

ESTRIDE = 272
PSTRIDE = 32

_mesh = plsc.VectorSubcoreMesh(core_axis_name="c", subcore_axis_name="s")


def _skew(a, n, stride):
    j = jnp.arange(n)
    out = jnp.zeros(a.shape[:-1] + (stride,), a.dtype)
    return out.at[..., j + (j // 16)].set(a)


@functools.partial(
    pl.kernel,
    mesh=_mesh,
    out_type=jax.ShapeDtypeStruct((B,), jnp.float32),
    compiler_params=pltpu.CompilerParams(needs_layout_passes=False),
    scratch_types=[
        pltpu.VMEM((F, RPT), jnp.float32),
        pltpu.VMEM((F * ESTRIDE,), jnp.float32),
        pltpu.VMEM((F * 256,), jnp.float32),
        pltpu.VMEM((P * 2 * PSTRIDE,), jnp.float32),
        pltpu.VMEM((P * 1024,), jnp.float32),
        pltpu.VMEM((2 * P, 16), jnp.int32),
        pltpu.VMEM((16,), jnp.float32),
        pltpu.VMEM((RPT,), jnp.float32),
        pltpu.SemaphoreType.DMA,
    ],
)
def _ebm_sc(xt_hbm, edges_hbm, v_hbm, pe_hbm, t2_hbm, pidx_hbm, bias_hbm,
            out_hbm,
            xt_v, edges_v, v_v, pe_v, t2_v, pidx_v, bias_v, out_v, sem):
    wid = lax.axis_index("s") * 2 + lax.axis_index("c")
    base = wid * RPT

    copies = [
        pltpu.async_copy(xt_hbm.at[:, pl.ds(base, RPT)], xt_v, sem),
        pltpu.async_copy(edges_hbm, edges_v, sem),
        pltpu.async_copy(v_hbm, v_v, sem),
        pltpu.async_copy(pe_hbm, pe_v, sem),
        pltpu.async_copy(t2_hbm, t2_v, sem),
        pltpu.async_copy(pidx_hbm, pidx_v, sem),
        pltpu.async_copy(bias_hbm, bias_v, sem),
    ]
    for c in copies:
        c.wait()

    lanes = lax.iota(jnp.int32, 16)
    zeros = jnp.zeros((16,), jnp.int32)
    bias_splat = bias_v[...]
    pid_l = [pidx_v[2 * p, :] for p in range(P)]
    pid_r = [pidx_v[2 * p + 1, :] for p in range(P)]

    def one_group(g):
        row0 = g * 16
        rowv = row0 + lanes
        acc = bias_splat
        for f in range(F):
            x = xt_v[f, pl.ds(row0, 16)]
            bsk = zeros
            for k in (128, 64, 32, 16, 8, 4, 2, 1):
                e = plsc.load_gather(
                    edges_v, [bsk + ((k - 1) + (k - 1) // 16 + f * ESTRIDE)])
                bsk = bsk + jnp.where(e <= x, k + (k >> 4), 0)
            b = bsk - ((bsk * 3856) >> 16)
            acc = acc + plsc.load_gather(v_v, [b + f * 256])
        for p in range(P):
            xl = plsc.load_gather(xt_v, [pid_l[p], rowv])
            xr = plsc.load_gather(xt_v, [pid_r[p], rowv])
            blsk = zeros
            brsk = zeros
            for k in (16, 8, 4, 2, 1):
                el = plsc.load_gather(
                    pe_v, [blsk + ((k - 1) + (k - 1) // 16 + (2 * p) * PSTRIDE)])
                blsk = blsk + jnp.where(el <= xl, k + (k >> 4), 0)
                er = plsc.load_gather(
                    pe_v, [brsk + ((k - 1) + (k - 1) // 16 + (2 * p + 1) * PSTRIDE)])
                brsk = brsk + jnp.where(er <= xr, k + (k >> 4), 0)
            bl = blsk - ((blsk * 3856) >> 16)
            br = brsk - ((brsk * 3856) >> 16)
            flat = (bl << 5) + br + p * 1024
            acc = acc + plsc.load_gather(t2_v, [flat])
        out_v[pl.ds(row0, 16)] = acc

    def body(i, carry):
        for u in range(UNROLL):
            one_group(i * UNROLL + u)
        return carry

    lax.fori_loop(0, NG // UNROLL, body, 0)
    pltpu.sync_copy(out_v, out_hbm.at[pl.ds(base, RPT)])


def kernel(X, edges, W, pair_edges, pair_tables, pair_idx, bias):
    xt = X.T
    edges_s = _skew(edges, E, ESTRIDE).reshape(-1)
    v_f = jnp.roll(W, -1, axis=1).reshape(-1)
    pe_s = _skew(pair_edges, PE, PSTRIDE).reshape(-1)
    t2_f = jnp.roll(jnp.roll(pair_tables, -1, axis=1), -1, axis=2).reshape(-1)
    pidx_s = jnp.broadcast_to(
        pair_idx.reshape(2 * P, 1).astype(jnp.int32), (2 * P, 16))
    bias16 = jnp.broadcast_to(bias.astype(jnp.float32), (16,))
    return _ebm_sc(xt, edges_s, v_f, pe_s, t2_f, pidx_s, bias16)
